# baseline (speedup 1.0000x reference)
import functools

import jax
import jax.numpy as jnp
from jax import lax
from jax.experimental import pallas as pl
from jax.experimental.pallas import tpu as pltpu
from jax.experimental.pallas import tpu_sc as plsc

NUM_FIELDS = 26
VOCAB = 100000
EMB_DIM = 16
DENSE_DIM = 13
BATCH = 4096
H1, H2 = 64, 32
BN_EPS = 1e-5

PLANES = NUM_FIELDS * EMB_DIM
_NC, _NS = 2, 16


_MAIN = 99968
_H0 = 390 * 128
_H1 = _MAIN - _H0
_TAIL0 = VOCAB - 128
_UNITS = PLANES + NUM_FIELDS
_U_PER_W = 14


def _sc_gather(tab2, lin_tab, tab_tail, lin_tail, ids):
    mesh = plsc.VectorSubcoreMesh(core_axis_name="c", subcore_axis_name="s")

    @functools.partial(
        pl.kernel,
        mesh=mesh,
        compiler_params=pltpu.CompilerParams(needs_layout_passes=False),
        out_type=(
            jax.ShapeDtypeStruct((PLANES, BATCH), jnp.float32),
            jax.ShapeDtypeStruct((NUM_FIELDS, BATCH), jnp.float32),
        ),
        scratch_types=[
            pltpu.VMEM((_H0,), jnp.float32),
            pltpu.VMEM((_H1,), jnp.float32),
            pltpu.VMEM((128,), jnp.float32),
            pltpu.VMEM((BATCH,), jnp.int32),
            pltpu.VMEM((BATCH,), jnp.float32),
            pltpu.SemaphoreType.DMA,
            pltpu.SemaphoreType.DMA,
        ],
    )
    def k(tab, ltab, ttail, ltail, ids_h, x_out, lin_out,
          buf_a, buf_b, tbuf, idx_v, val_v, sem_a, sem_b):
        wid = lax.axis_index("s") * _NC + lax.axis_index("c")
        last = _UNITS - 1

        def fire_h0(un):
            @pl.when(un < PLANES)
            def _():
                pltpu.async_copy(tab.at[un, pl.ds(0, _H0)], buf_a, sem_a)

            @pl.when(un >= PLANES)
            def _():
                pltpu.async_copy(ltab.at[un - PLANES, pl.ds(0, _H0)],
                                 buf_a, sem_a)

        fire_h0(jnp.minimum(wid * _U_PER_W, last))

        def unit_body(j, _):
            u = wid * _U_PER_W + j
            valid = u < _UNITS
            um = jnp.minimum(u, last)
            is_fm = um < PLANES
            f = jnp.where(is_fm, um // EMB_DIM, um - PLANES)

            @pl.when(is_fm)
            def _():
                pltpu.async_copy(tab.at[um, pl.ds(_H0, _H1)], buf_b, sem_b)
                pltpu.async_copy(ttail.at[um], tbuf, sem_b)

            @pl.when(jnp.logical_not(is_fm))
            def _():
                lf = um - PLANES
                pltpu.async_copy(ltab.at[lf, pl.ds(_H0, _H1)], buf_b,
                                 sem_b)
                pltpu.async_copy(ltail.at[lf], tbuf, sem_b)

            pltpu.sync_copy(ids_h.at[pl.ds(f * BATCH, BATCH)], idx_v)

            pltpu.make_async_copy(tab.at[0, pl.ds(0, _H0)], buf_a,
                                  sem_a).wait()

            @pl.when(valid)
            def _():
                for g in range(BATCH // 16):
                    v = idx_v[pl.ds(g * 16, 16)]
                    v0 = jnp.minimum(v, _H0 - 1)
                    val_v[pl.ds(g * 16, 16)] = jnp.where(
                        v < _H0, plsc.load_gather(buf_a, [v0]), 0.0)

            fire_h0(jnp.minimum(u + 1, last))

            pltpu.make_async_copy(tab.at[0, pl.ds(_H0, _H1)], buf_b,
                                  sem_b).wait()
            pltpu.make_async_copy(ttail.at[0], tbuf, sem_b).wait()

            @pl.when(valid)
            def _():
                for g in range(BATCH // 16):
                    v = idx_v[pl.ds(g * 16, 16)]
                    v1 = jnp.clip(v - _H0, 0, _H1 - 1)
                    vt = jnp.maximum(v - _TAIL0, 0)
                    hi = jnp.where(v >= _MAIN,
                                   plsc.load_gather(tbuf, [vt]),
                                   plsc.load_gather(buf_b, [v1]))
                    cur = val_v[pl.ds(g * 16, 16)]
                    val_v[pl.ds(g * 16, 16)] = jnp.where(
                        v < _H0, cur, hi)

                @pl.when(is_fm)
                def _():
                    pltpu.sync_copy(val_v, x_out.at[um])

                @pl.when(jnp.logical_not(is_fm))
                def _():
                    pltpu.sync_copy(val_v, lin_out.at[um - PLANES])

            return _

        lax.fori_loop(0, _U_PER_W, unit_body, None)
        pltpu.make_async_copy(tab.at[0, pl.ds(0, _H0)], buf_a,
                              sem_a).wait()

    return k(tab2, lin_tab, tab_tail, lin_tail, ids)


def _tc_body(x_ref, d_ref, lin_ref, wd_ref, w1_ref, s1_ref, f1_ref,
             w2_ref, s2_ref, f2_ref, wout_ref, cbias_ref, out_ref):
    f32 = jnp.float32
    x = x_ref[...]
    d = d_ref[...]
    rows = lax.broadcasted_iota(jnp.int32, (EMB_DIM, PLANES), 0)
    cols = lax.broadcasted_iota(jnp.int32, (EMB_DIM, PLANES), 1)
    sel = jnp.where((cols & (EMB_DIM - 1)) == rows, 1.0, 0.0).astype(f32)

    sv = jnp.dot(sel, x, preferred_element_type=f32)
    sq = jnp.dot(sel, x * x, preferred_element_type=f32)
    fm_logit = 0.5 * jnp.sum(sv * sv - sq, axis=0, keepdims=True)

    lin_logit = jnp.sum(lin_ref[...], axis=0, keepdims=True)
    lin_logit = lin_logit + jnp.dot(wd_ref[...], d,
                                    preferred_element_type=f32)

    w1 = w1_ref[...]
    z = jnp.dot(w1[:, :PLANES], x, preferred_element_type=f32)
    z = z + jnp.dot(w1[:, PLANES:], d, preferred_element_type=f32)
    h = jnp.maximum(z * s1_ref[...] + f1_ref[...], 0.0)
    z2 = jnp.dot(w2_ref[...], h, preferred_element_type=f32)
    h2 = jnp.maximum(z2 * s2_ref[...] + f2_ref[...], 0.0)
    dnn_logit = jnp.dot(wout_ref[...], h2, preferred_element_type=f32)

    total = lin_logit + fm_logit + dnn_logit + cbias_ref[...]
    out_ref[...] = jax.nn.sigmoid(total)


def _tc_dense(x_t, d_t, lin_t, wd, w1, s1, f1, w2, s2, f2, wout, cbias):
    bm = 1024
    grid = (BATCH // bm,)
    full = lambda shape: pl.BlockSpec(shape, lambda i: (0,) * len(shape))
    col = lambda rows: pl.BlockSpec((rows, bm), lambda i: (0, i))
    return pl.pallas_call(
        _tc_body,
        grid=grid,
        in_specs=[
            col(PLANES),
            col(DENSE_DIM),
            col(NUM_FIELDS),
            full((1, DENSE_DIM)),
            full((H1, PLANES + DENSE_DIM)),
            full((H1, 1)), full((H1, 1)),
            full((H2, H1)),
            full((H2, 1)), full((H2, 1)),
            full((1, H2)),
            full((1, 1)),
        ],
        out_specs=col(1),
        out_shape=jax.ShapeDtypeStruct((1, BATCH), jnp.float32),
    )(x_t, d_t, lin_t, wd, w1, s1, f1, w2, s2, f2, wout, cbias)


def kernel(sparse_inputs, dense_inputs, fm_tables, lin_tables, W_dense,
           b_dense, bias, W1, b1, g1, bt1, W2, b2, g2, bt2, Wout, bout):
    i32 = jnp.int32
    vT = sparse_inputs.astype(i32).T
    d_t = dense_inputs.T
    tab2 = fm_tables.transpose(0, 2, 1).reshape(PLANES, VOCAB)
    tab_tail = tab2[:, _TAIL0:]
    lin_tail = lin_tables[:, _TAIL0:]

    x_t, lin_t = _sc_gather(tab2, lin_tables, tab_tail, lin_tail,
                            vT.reshape(-1))

    inv = lax.rsqrt(jnp.float32(1.0 + BN_EPS))
    s1 = (g1 * inv).reshape(H1, 1)
    f1 = (b1 * g1 * inv + bt1).reshape(H1, 1)
    s2 = (g2 * inv).reshape(H2, 1)
    f2 = (b2 * g2 * inv + bt2).reshape(H2, 1)
    cbias = (bias + b_dense + bout).reshape(1, 1)
    out = _tc_dense(x_t, d_t, lin_t, W_dense, W1, s1, f1, W2, s2, f2,
                    Wout, cbias)
    return out.reshape(BATCH)


# speedup vs baseline: 1.2057x; 1.2057x over previous
import functools

import jax
import jax.numpy as jnp
from jax import lax
from jax.experimental import pallas as pl
from jax.experimental.pallas import tpu as pltpu
from jax.experimental.pallas import tpu_sc as plsc

NUM_FIELDS = 26
VOCAB = 100000
EMB_DIM = 16
DENSE_DIM = 13
BATCH = 4096
H1, H2 = 64, 32
BN_EPS = 1e-5

PLANES = NUM_FIELDS * EMB_DIM
_NC, _NS = 2, 16


_MAIN = 99968
_TAIL0 = VOCAB - 128
_UNITS = PLANES + NUM_FIELDS
_U_PER_W = 14


def _sc_gather(tab2, lin_tab, tab_tail, lin_tail, ids):
    mesh = plsc.VectorSubcoreMesh(core_axis_name="c", subcore_axis_name="s")

    @functools.partial(
        pl.kernel,
        mesh=mesh,
        compiler_params=pltpu.CompilerParams(needs_layout_passes=False),
        out_type=(
            jax.ShapeDtypeStruct((PLANES, BATCH), jnp.float32),
            jax.ShapeDtypeStruct((NUM_FIELDS, BATCH), jnp.float32),
        ),
        scratch_types=[
            pltpu.VMEM((_MAIN,), jnp.float32),
            pltpu.VMEM((128,), jnp.float32),
            pltpu.VMEM((BATCH,), jnp.int32),
            pltpu.VMEM((BATCH,), jnp.float32),
            pltpu.SemaphoreType.DMA,
        ],
    )
    def k(tab, ltab, ttail, ltail, ids_h, x_out, lin_out,
          buf, tbuf, idx_v, val_v, sem):
        wid = lax.axis_index("s") * _NC + lax.axis_index("c")

        def unit_body(j, _):
            u = wid * _U_PER_W + j

            @pl.when(u < _UNITS)
            def _():
                is_fm = u < PLANES
                f = jnp.where(is_fm, u // EMB_DIM, u - PLANES)

                @pl.when(is_fm)
                def _():
                    cp = pltpu.async_copy(
                        tab.at[u, pl.ds(0, _MAIN)], buf, sem)
                    cpt = pltpu.async_copy(ttail.at[u], tbuf, sem)
                    pltpu.sync_copy(
                        ids_h.at[pl.ds(f * BATCH, BATCH)], idx_v)
                    cpt.wait()
                    cp.wait()

                @pl.when(jnp.logical_not(is_fm))
                def _():
                    lf = u - PLANES
                    cp = pltpu.async_copy(
                        ltab.at[lf, pl.ds(0, _MAIN)], buf, sem)
                    cpt = pltpu.async_copy(ltail.at[lf], tbuf, sem)
                    pltpu.sync_copy(
                        ids_h.at[pl.ds(f * BATCH, BATCH)], idx_v)
                    cpt.wait()
                    cp.wait()

                for g in range(BATCH // 16):
                    v = idx_v[pl.ds(g * 16, 16)]
                    vmain = jnp.minimum(v, _MAIN - 1)
                    vtail = jnp.maximum(v - _TAIL0, 0)
                    vals = jnp.where(
                        v < _MAIN,
                        plsc.load_gather(buf, [vmain]),
                        plsc.load_gather(tbuf, [vtail]))
                    val_v[pl.ds(g * 16, 16)] = vals

                @pl.when(is_fm)
                def _():
                    pltpu.sync_copy(val_v, x_out.at[u])

                @pl.when(jnp.logical_not(is_fm))
                def _():
                    pltpu.sync_copy(val_v, lin_out.at[u - PLANES])

            return _

        lax.fori_loop(0, _U_PER_W, unit_body, None)

    return k(tab2, lin_tab, tab_tail, lin_tail, ids)


def _tc_body(x_ref, d_ref, lin_ref, wd_ref, w1_ref, s1_ref, f1_ref,
             w2_ref, s2_ref, f2_ref, wout_ref, cbias_ref, out_ref):
    f32 = jnp.float32
    x = x_ref[...]
    d = d_ref[...]
    rows = lax.broadcasted_iota(jnp.int32, (EMB_DIM, PLANES), 0)
    cols = lax.broadcasted_iota(jnp.int32, (EMB_DIM, PLANES), 1)
    sel = jnp.where((cols & (EMB_DIM - 1)) == rows, 1.0, 0.0).astype(f32)

    sv = jnp.dot(sel, x, preferred_element_type=f32)
    sq = jnp.dot(sel, x * x, preferred_element_type=f32)
    fm_logit = 0.5 * jnp.sum(sv * sv - sq, axis=0, keepdims=True)

    lin_logit = jnp.sum(lin_ref[...], axis=0, keepdims=True)
    lin_logit = lin_logit + jnp.dot(wd_ref[...], d,
                                    preferred_element_type=f32)

    w1 = w1_ref[...]
    z = jnp.dot(w1[:, :PLANES], x, preferred_element_type=f32)
    z = z + jnp.dot(w1[:, PLANES:], d, preferred_element_type=f32)
    h = jnp.maximum(z * s1_ref[...] + f1_ref[...], 0.0)
    z2 = jnp.dot(w2_ref[...], h, preferred_element_type=f32)
    h2 = jnp.maximum(z2 * s2_ref[...] + f2_ref[...], 0.0)
    dnn_logit = jnp.dot(wout_ref[...], h2, preferred_element_type=f32)

    total = lin_logit + fm_logit + dnn_logit + cbias_ref[...]
    out_ref[...] = jax.nn.sigmoid(total)


def _tc_dense(x_t, d_t, lin_t, wd, w1, s1, f1, w2, s2, f2, wout, cbias):
    bm = 1024
    grid = (BATCH // bm,)
    full = lambda shape: pl.BlockSpec(shape, lambda i: (0,) * len(shape))
    col = lambda rows: pl.BlockSpec((rows, bm), lambda i: (0, i))
    return pl.pallas_call(
        _tc_body,
        grid=grid,
        in_specs=[
            col(PLANES),
            col(DENSE_DIM),
            col(NUM_FIELDS),
            full((1, DENSE_DIM)),
            full((H1, PLANES + DENSE_DIM)),
            full((H1, 1)), full((H1, 1)),
            full((H2, H1)),
            full((H2, 1)), full((H2, 1)),
            full((1, H2)),
            full((1, 1)),
        ],
        out_specs=col(1),
        out_shape=jax.ShapeDtypeStruct((1, BATCH), jnp.float32),
    )(x_t, d_t, lin_t, wd, w1, s1, f1, w2, s2, f2, wout, cbias)


def kernel(sparse_inputs, dense_inputs, fm_tables, lin_tables, W_dense,
           b_dense, bias, W1, b1, g1, bt1, W2, b2, g2, bt2, Wout, bout):
    i32 = jnp.int32
    vT = sparse_inputs.astype(i32).T
    d_t = dense_inputs.T
    tab2 = fm_tables.transpose(0, 2, 1).reshape(PLANES, VOCAB)
    tab_tail = tab2[:, _TAIL0:]
    lin_tail = lin_tables[:, _TAIL0:]

    x_t, lin_t = _sc_gather(tab2, lin_tables, tab_tail, lin_tail,
                            vT.reshape(-1))

    inv = lax.rsqrt(jnp.float32(1.0 + BN_EPS))
    s1 = (g1 * inv).reshape(H1, 1)
    f1 = (b1 * g1 * inv + bt1).reshape(H1, 1)
    s2 = (g2 * inv).reshape(H2, 1)
    f2 = (b2 * g2 * inv + bt2).reshape(H2, 1)
    cbias = (bias + b_dense + bout).reshape(1, 1)
    out = _tc_dense(x_t, d_t, lin_t, W_dense, W1, s1, f1, W2, s2, f2,
                    Wout, cbias)
    return out.reshape(BATCH)
